# baseline (speedup 1.0000x reference)
import functools

import jax
import jax.numpy as jnp
from jax import lax
from jax.experimental import pallas as pl
from jax.experimental.pallas import tpu as pltpu
from jax.experimental.pallas import tpu_sc as plsc

D = 64
BATCH = 4096
SEQ = 200
L = 16
NC, NS = 2, 16
NW = NC * NS
BBLK = BATCH // NW
NGRP = BBLK // L
EPS = 1e-5


def _rsqrt(v):
    bits = lax.bitcast_convert_type(v, jnp.int32)
    y = lax.bitcast_convert_type(
        0x5F3759DF - lax.shift_right_logical(bits, 1), jnp.float32)
    h = 0.5 * v
    y = y * (1.5 - h * y * y)
    y = y * (1.5 - h * y * y)
    return y


def _make_kernel():
    mesh = plsc.VectorSubcoreMesh(
        core_axis_name="c", subcore_axis_name="s",
        num_cores=NC, num_subcores=NS)

    @functools.partial(
        pl.kernel, mesh=mesh,
        compiler_params=pltpu.CompilerParams(
            needs_layout_passes=False, use_tc_tiling_on_sc=False),
        out_type=jax.ShapeDtypeStruct((SEQ, D // 8, NW, 8, BBLK), jnp.float32),
        scratch_types=[
            pltpu.VMEM((SEQ, BBLK), jnp.int32),
            pltpu.VMEM((SEQ, BBLK), jnp.int32),
            pltpu.VMEM((BBLK, 2 * D), jnp.float32),
            pltpu.VMEM((BBLK, 2 * D), jnp.float32),
            pltpu.VMEM((D // 8, 8, BBLK), jnp.float32),
            pltpu.VMEM((D // 8, 8, BBLK), jnp.float32),
            pltpu.VMEM((D,), jnp.float32),
            pltpu.VMEM((D,), jnp.float32),
            pltpu.SemaphoreType.DMA,
            pltpu.SemaphoreType.DMA,
            pltpu.SemaphoreType.DMA,
            pltpu.SemaphoreType.DMA,
        ],
    )
    def k(xt_hbm, table2_hbm, w_hbm, bias_hbm, out_hbm,
          idx_v, idxp_v, rows0_v, rows1_v, outb0_v, outb1_v, w_v, b_v,
          sg0, sg1, sw0, sw1):
        wid = lax.axis_index("s") * NC + lax.axis_index("c")

        pltpu.sync_copy(w_hbm, w_v)
        pltpu.sync_copy(bias_hbm, b_v)
        pltpu.sync_copy(xt_hbm.at[:, pl.ds(wid * BBLK, BBLK)], idx_v)

        def shift_body(i, _):
            for c in range(NGRP):
                idxp_v[i, pl.ds(c * L, L)] = lax.shift_right_logical(
                    idx_v[i, pl.ds(c * L, L)], 1)
            return 0
        lax.fori_loop(0, SEQ, shift_body, 0)

        rows_bufs = (rows0_v, rows1_v)
        out_bufs = (outb0_v, outb1_v)
        sems_g = (sg0, sg1)
        sems_w = (sw0, sw1)

        def gather(buf, g, wait):
            d = pltpu.make_async_copy(
                table2_hbm.at[idxp_v.at[g]], rows_bufs[buf], sems_g[buf])
            d.wait() if wait else d.start()

        def write(buf, g, wait):
            d = pltpu.make_async_copy(
                out_bufs[buf], out_hbm.at[g, :, wid], sems_w[buf])
            d.wait() if wait else d.start()

        w_vecs = [w_v[pl.ds(k * L, L)] for k in range(D // L)]
        b_vecs = [b_v[pl.ds(k * L, L)] for k in range(D // L)]
        w_sc = [w_vecs[c // L][c % L] for c in range(D)]
        b_sc = [b_vecs[c // L][c % L] for c in range(D)]
        lane = jnp.arange(L, dtype=jnp.int32)

        def compute(buf, g):
            rows_ref = rows_bufs[buf]
            outb_ref = out_bufs[buf]

            def grp_body(grp, _):
                raw = idx_v[g, pl.ds(grp * L, L)]
                col0 = lax.shift_left(lax.bitwise_and(raw, 1), 6)
                ridx = grp * L + lane
                acc_s = jnp.zeros((L,), jnp.float32)
                acc_q = jnp.zeros((L,), jnp.float32)
                cidx = col0
                for d in range(D):
                    v = plsc.load_gather(rows_ref, [ridx, cidx])
                    acc_s = acc_s + v
                    acc_q = acc_q + v * v
                    cidx = cidx + 1
                mean = acc_s * (1.0 / D)
                var = acc_q * (1.0 / D) - mean * mean + EPS
                rstd = _rsqrt(var)
                cidx = col0
                for d in range(D):
                    v = plsc.load_gather(rows_ref, [ridx, cidx])
                    rw = rstd * w_sc[d]
                    off = b_sc[d] - mean * rw
                    outb_ref[d // 8, d % 8, pl.ds(grp * L, L)] = v * rw + off
                    cidx = cidx + 1
                return 0
            lax.fori_loop(0, NGRP, grp_body, 0)

        gather(0, 0, False)
        gather(1, 1, False)

        def tile_body(ss, _):
            for b in range(2):
                g = 2 * ss + b
                gather(b, g, True)

                @pl.when(ss >= 1)
                def _():
                    write(b, g - 2, True)

                compute(b, g)

                @pl.when(ss < SEQ // 2 - 1)
                def _():
                    gather(b, g + 2, False)

                write(b, g, False)
            return 0

        lax.fori_loop(0, SEQ // 2, tile_body, 0)

        write(0, SEQ - 2, True)
        write(1, SEQ - 1, True)

    return k


_kernel = _make_kernel()


@jax.jit
def kernel(x, table, ln_weight, ln_bias):
    xt = jnp.transpose(x.astype(jnp.int32), (1, 0))
    table2 = table.reshape(table.shape[0] // 2, 2 * D)
    out5d = _kernel(xt, table2, ln_weight, ln_bias)
    return jnp.transpose(out5d, (2, 4, 0, 1, 3)).reshape(BATCH, SEQ, D)
